# Initial kernel scaffold; baseline (speedup 1.0000x reference)
import jax
import jax.numpy as jnp
from jax.experimental import pallas as pl


def kernel(x, glove_table, W, b):
    raise NotImplementedError("write your pallas kernel here")


# speedup vs baseline: 1.1612x; 1.1612x over previous
import functools

import jax
import jax.numpy as jnp
from jax import lax
from jax.experimental import pallas as pl
from jax.experimental.pallas import tpu as pltpu
from jax.experimental.pallas import tpu_sc as plsc

_GDIM = 300
_GPAD = 384
_DMODEL = 768

_NC = 2
_NS = 16
_NW = _NC * _NS

_CHUNK = 80


def _sc_gather_build(n_rows: int):
    b_per_w = n_rows // _NW
    assert n_rows % _NW == 0 and b_per_w % _CHUNK == 0
    n_chunks = b_per_w // _CHUNK

    mesh = plsc.VectorSubcoreMesh(core_axis_name="c", subcore_axis_name="s")

    @functools.partial(
        pl.kernel,
        mesh=mesh,
        out_type=jax.ShapeDtypeStruct((n_rows, _GPAD), jnp.float32),
        scratch_types=[
            pltpu.VMEM((b_per_w,), jnp.int32),
            pltpu.VMEM((_CHUNK, _GPAD), jnp.float32),
            pltpu.VMEM((_CHUNK, _GPAD), jnp.float32),
            pltpu.SemaphoreType.DMA,
            pltpu.SemaphoreType.DMA,
        ],
    )
    def sc_gather(table_hbm, idx_hbm, out_hbm, idx_v, buf0, buf1, sem0, sem1):
        wid = lax.axis_index("s") * _NC + lax.axis_index("c")
        base = wid * b_per_w
        pltpu.sync_copy(idx_hbm.at[pl.ds(base, b_per_w)], idx_v)
        bufs = (buf0, buf1)
        sems = (sem0, sem1)
        copies = [None] * n_chunks
        copies[0] = pltpu.async_copy(
            table_hbm.at[idx_v.at[pl.ds(0, _CHUNK)]], bufs[0], sems[0])
        for i in range(n_chunks):
            if i + 1 < n_chunks:
                j = i + 1
                copies[j] = pltpu.async_copy(
                    table_hbm.at[idx_v.at[pl.ds(j * _CHUNK, _CHUNK)]],
                    bufs[j % 2], sems[j % 2])
            copies[i].wait()
            pltpu.sync_copy(
                bufs[i % 2], out_hbm.at[pl.ds(base + i * _CHUNK, _CHUNK)])

    return sc_gather


def _tc_project_build(n_rows: int, bm: int):
    assert n_rows % bm == 0

    def body(g_ref, w_ref, b_ref, o_ref):
        o_ref[...] = (
            jnp.dot(g_ref[...], w_ref[...],
                    preferred_element_type=jnp.float32)
            + b_ref[...]
        )

    return pl.pallas_call(
        body,
        grid=(n_rows // bm,),
        in_specs=[
            pl.BlockSpec((bm, _GPAD), lambda i: (i, 0)),
            pl.BlockSpec((_GPAD, _DMODEL), lambda i: (0, 0)),
            pl.BlockSpec((1, _DMODEL), lambda i: (0, 0)),
        ],
        out_specs=pl.BlockSpec((bm, _DMODEL), lambda i: (i, 0)),
        out_shape=jax.ShapeDtypeStruct((n_rows, _DMODEL), jnp.float32),
    )


def kernel(x, glove_table, W, b):
    batch, hist = x.shape
    n_rows = batch * hist
    idx = x.astype(jnp.int32).reshape(-1)
    table_p = jnp.pad(glove_table, ((0, 0), (0, _GPAD - _GDIM)))
    w_p = jnp.pad(W, ((0, _GPAD - _GDIM), (0, 0)))
    gathered = _sc_gather_build(n_rows)(table_p, idx)
    out = _tc_project_build(n_rows, 512)(
        gathered, w_p, b.reshape(1, _DMODEL))
    return out.reshape(batch, hist, _DMODEL)


# speedup vs baseline: 3.9465x; 3.3985x over previous
import functools

import jax
import jax.numpy as jnp
from jax import lax
from jax.experimental import pallas as pl
from jax.experimental.pallas import tpu as pltpu
from jax.experimental.pallas import tpu_sc as plsc

_GDIM = 300
_DMODEL = 768

_NC = 2
_NS = 16
_NW = _NC * _NS

_CHUNK = 64


def _sc_gather_build(n_rows: int, width: int):
    b_per_w = n_rows // _NW
    assert n_rows % _NW == 0 and b_per_w % _CHUNK == 0
    n_chunks = b_per_w // _CHUNK

    mesh = plsc.VectorSubcoreMesh(core_axis_name="c", subcore_axis_name="s")

    @functools.partial(
        pl.kernel,
        mesh=mesh,
        out_type=jax.ShapeDtypeStruct((n_rows, width), jnp.float32),
        scratch_types=[
            pltpu.VMEM((b_per_w,), jnp.int32),
            pltpu.VMEM((_CHUNK, width), jnp.float32),
            pltpu.VMEM((_CHUNK, width), jnp.float32),
            pltpu.SemaphoreType.DMA,
            pltpu.SemaphoreType.DMA,
        ],
    )
    def sc_gather(table_hbm, idx_hbm, out_hbm, idx_v, buf0, buf1, sem0, sem1):
        wid = lax.axis_index("s") * _NC + lax.axis_index("c")
        base = wid * b_per_w
        pltpu.sync_copy(idx_hbm.at[pl.ds(base, b_per_w)], idx_v)
        bufs = (buf0, buf1)
        sems = (sem0, sem1)
        copies = [None] * n_chunks
        copies[0] = pltpu.async_copy(
            table_hbm.at[idx_v.at[pl.ds(0, _CHUNK)]], bufs[0], sems[0])
        for i in range(n_chunks):
            if i + 1 < n_chunks:
                j = i + 1
                copies[j] = pltpu.async_copy(
                    table_hbm.at[idx_v.at[pl.ds(j * _CHUNK, _CHUNK)]],
                    bufs[j % 2], sems[j % 2])
            copies[i].wait()
            pltpu.sync_copy(
                bufs[i % 2], out_hbm.at[pl.ds(base + i * _CHUNK, _CHUNK)])

    return sc_gather


def _tc_project_table_build(vocab: int, bv: int):
    grid = (vocab + bv - 1) // bv

    def body(tT_ref, w_ref, b_ref, o_ref):
        o_ref[...] = (
            jax.lax.dot_general(
                tT_ref[...], w_ref[...],
                dimension_numbers=(((0,), (0,)), ((), ())),
                preferred_element_type=jnp.float32)
            + b_ref[...]
        )

    return pl.pallas_call(
        body,
        grid=(grid,),
        in_specs=[
            pl.BlockSpec((_GDIM, bv), lambda i: (0, i)),
            pl.BlockSpec((_GDIM, _DMODEL), lambda i: (0, 0)),
            pl.BlockSpec((1, _DMODEL), lambda i: (0, 0)),
        ],
        out_specs=pl.BlockSpec((bv, _DMODEL), lambda i: (i, 0)),
        out_shape=jax.ShapeDtypeStruct((vocab, _DMODEL), jnp.float32),
    )


def kernel(x, glove_table, W, b):
    batch, hist = x.shape
    vocab = glove_table.shape[0]
    n_rows = batch * hist
    idx = x.T.astype(jnp.int32).reshape(-1)
    table_t = glove_table.T
    proj = _tc_project_table_build(vocab, 1024)(
        table_t, W, b.reshape(1, _DMODEL))
    out_tm = _sc_gather_build(n_rows, _DMODEL)(proj, idx)
    return out_tm.reshape(hist, batch, _DMODEL).transpose(1, 0, 2)


# speedup vs baseline: 4.4080x; 1.1169x over previous
import functools

import jax
import jax.numpy as jnp
from jax import lax
from jax.experimental import pallas as pl
from jax.experimental.pallas import tpu as pltpu
from jax.experimental.pallas import tpu_sc as plsc

_GDIM = 300
_DMODEL = 768

_NC = 2
_NS = 16
_NW = _NC * _NS

_CHUNK = 64


def _sc_gather_build(n_rows: int, width: int):
    b_per_w = n_rows // _NW
    assert n_rows % _NW == 0 and b_per_w % _CHUNK == 0
    n_chunks = b_per_w // _CHUNK

    mesh = plsc.VectorSubcoreMesh(core_axis_name="c", subcore_axis_name="s")

    @functools.partial(
        pl.kernel,
        mesh=mesh,
        out_type=jax.ShapeDtypeStruct((n_rows, width), jnp.float32),
        scratch_types=[
            pltpu.VMEM((b_per_w,), jnp.int32),
            pltpu.VMEM((_CHUNK, width), jnp.float32),
            pltpu.VMEM((_CHUNK, width), jnp.float32),
            pltpu.SemaphoreType.DMA,
            pltpu.SemaphoreType.DMA,
        ],
    )
    def sc_gather(table_hbm, idx_hbm, out_hbm, idx_v, buf0, buf1, sem0, sem1):
        wid = lax.axis_index("s") * _NC + lax.axis_index("c")
        base = wid * b_per_w
        pltpu.sync_copy(idx_hbm.at[pl.ds(base, b_per_w)], idx_v)
        bufs = (buf0, buf1)
        sems = (sem0, sem1)
        copies = [None] * n_chunks
        copies[0] = pltpu.async_copy(
            table_hbm.at[idx_v.at[pl.ds(0, _CHUNK)]], bufs[0], sems[0])
        for i in range(n_chunks):
            if i + 1 < n_chunks:
                j = i + 1
                copies[j] = pltpu.async_copy(
                    table_hbm.at[idx_v.at[pl.ds(j * _CHUNK, _CHUNK)]],
                    bufs[j % 2], sems[j % 2])
            copies[i].wait()
            pltpu.sync_copy(
                bufs[i % 2], out_hbm.at[pl.ds(base + i * _CHUNK, _CHUNK)])

    return sc_gather


def _tc_project_table_build(vocab: int, bv: int):
    grid = (vocab + bv - 1) // bv

    def body(tT_ref, w_ref, b_ref, o_ref):
        o_ref[...] = (
            jax.lax.dot_general(
                tT_ref[...].astype(jnp.bfloat16),
                w_ref[...].astype(jnp.bfloat16),
                dimension_numbers=(((0,), (0,)), ((), ())),
                preferred_element_type=jnp.float32)
            + b_ref[...]
        )

    return pl.pallas_call(
        body,
        grid=(grid,),
        in_specs=[
            pl.BlockSpec((_GDIM, bv), lambda i: (0, i)),
            pl.BlockSpec((_GDIM, _DMODEL), lambda i: (0, 0)),
            pl.BlockSpec((1, _DMODEL), lambda i: (0, 0)),
        ],
        out_specs=pl.BlockSpec((bv, _DMODEL), lambda i: (i, 0)),
        out_shape=jax.ShapeDtypeStruct((vocab, _DMODEL), jnp.float32),
    )


def kernel(x, glove_table, W, b):
    batch, hist = x.shape
    vocab = glove_table.shape[0]
    n_rows = batch * hist
    idx = x.T.astype(jnp.int32).reshape(-1)
    table_t = glove_table.T
    proj = _tc_project_table_build(vocab, 2048)(
        table_t, W, b.reshape(1, _DMODEL))
    out_tm = _sc_gather_build(n_rows, _DMODEL)(proj, idx)
    return out_tm.reshape(hist, batch, _DMODEL).transpose(1, 0, 2)


# speedup vs baseline: 4.5603x; 1.0345x over previous
import functools

import jax
import jax.numpy as jnp
from jax import lax
from jax.experimental import pallas as pl
from jax.experimental.pallas import tpu as pltpu
from jax.experimental.pallas import tpu_sc as plsc

_GDIM = 300
_DMODEL = 768

_NC = 2
_NS = 16
_NW = _NC * _NS

_CHUNK = 80


def _sc_gather_build(n_rows: int, width: int):
    b_per_w = n_rows // _NW
    assert n_rows % _NW == 0 and b_per_w % _CHUNK == 0
    n_chunks = b_per_w // _CHUNK

    mesh = plsc.VectorSubcoreMesh(core_axis_name="c", subcore_axis_name="s")

    @functools.partial(
        pl.kernel,
        mesh=mesh,
        out_type=jax.ShapeDtypeStruct((n_rows, width), jnp.float32),
        scratch_types=[
            pltpu.VMEM((b_per_w,), jnp.int32),
            pltpu.VMEM((_CHUNK, width), jnp.float32),
            pltpu.VMEM((_CHUNK, width), jnp.float32),
            pltpu.SemaphoreType.DMA,
            pltpu.SemaphoreType.DMA,
        ],
    )
    def sc_gather(table_hbm, idx_hbm, out_hbm, idx_v, buf0, buf1, sem0, sem1):
        wid = lax.axis_index("s") * _NC + lax.axis_index("c")
        base = wid * b_per_w
        pltpu.sync_copy(idx_hbm.at[pl.ds(base, b_per_w)], idx_v)
        bufs = (buf0, buf1)
        sems = (sem0, sem1)
        copies = [None] * n_chunks
        copies[0] = pltpu.async_copy(
            table_hbm.at[idx_v.at[pl.ds(0, _CHUNK)]], bufs[0], sems[0])
        for i in range(n_chunks):
            if i + 1 < n_chunks:
                j = i + 1
                copies[j] = pltpu.async_copy(
                    table_hbm.at[idx_v.at[pl.ds(j * _CHUNK, _CHUNK)]],
                    bufs[j % 2], sems[j % 2])
            copies[i].wait()
            pltpu.sync_copy(
                bufs[i % 2], out_hbm.at[pl.ds(base + i * _CHUNK, _CHUNK)])

    return sc_gather


def _tc_project_table_build(vocab: int, bv: int):
    grid = (vocab + bv - 1) // bv

    def body(tT_ref, w_ref, b_ref, o_ref):
        o_ref[...] = (
            jax.lax.dot_general(
                tT_ref[...].astype(jnp.bfloat16),
                w_ref[...].astype(jnp.bfloat16),
                dimension_numbers=(((0,), (0,)), ((), ())),
                preferred_element_type=jnp.float32)
            + b_ref[...]
        )

    return pl.pallas_call(
        body,
        grid=(grid,),
        in_specs=[
            pl.BlockSpec((_GDIM, bv), lambda i: (0, i)),
            pl.BlockSpec((_GDIM, _DMODEL), lambda i: (0, 0)),
            pl.BlockSpec((1, _DMODEL), lambda i: (0, 0)),
        ],
        out_specs=pl.BlockSpec((bv, _DMODEL), lambda i: (i, 0)),
        out_shape=jax.ShapeDtypeStruct((vocab, _DMODEL), jnp.float32),
    )


def kernel(x, glove_table, W, b):
    batch, hist = x.shape
    vocab = glove_table.shape[0]
    n_rows = batch * hist
    idx = x.T.astype(jnp.int32).reshape(-1)
    table_t = glove_table.T
    proj = _tc_project_table_build(vocab, 4096)(
        table_t, W, b.reshape(1, _DMODEL))
    out_tm = _sc_gather_build(n_rows, _DMODEL)(proj, idx)
    return out_tm.reshape(hist, batch, _DMODEL).transpose(1, 0, 2)
